# baseline (speedup 1.0000x reference)
import functools

import jax
import jax.numpy as jnp
from jax.experimental import pallas as pl

_WPAD = 40


def _rpn_body(x0_ref, x1_ref, x2_ref, w1_ref, b1_ref, w2_ref, b2_ref, o_ref,
              *, rt):
    r0 = pl.multiple_of(pl.program_id(1) * rt, 8)
    xrefs = (x0_ref, x1_ref, x2_ref)
    acc = jnp.zeros((rt, w1_ref.shape[2]), jnp.float32)
    for t in range(9):
        ky, kx = divmod(t, 3)
        xs = xrefs[kx][0, pl.ds(r0 + ky * _WPAD, rt), :]
        acc = acc + jnp.dot(xs, w1_ref[t], preferred_element_type=jnp.float32)
    h = jnp.maximum(acc + b1_ref[0].astype(jnp.float32), 0.0).astype(jnp.bfloat16)
    out = jnp.dot(h, w2_ref[...], preferred_element_type=jnp.float32)
    o_ref[0] = out + b2_ref[0].astype(jnp.float32)


def kernel(feats, W1, b1, W2, b2):
    B, C, H, W = feats.shape
    dim = W1.shape[0]
    co = W2.shape[0]
    Hp = H + 2

    RT = 128
    nR = -(-(H * _WPAD) // RT)
    Rpad = nR * RT + 2 * _WPAD
    CO2 = 128

    x = jnp.transpose(feats, (0, 2, 3, 1))
    xp = jnp.pad(x, ((0, 0), (1, 1), (1, _WPAD - W - 1), (0, 0)))
    xf = xp.reshape(B, Hp * _WPAD, C)
    xf = jnp.pad(xf, ((0, 0), (0, Rpad + 2 - Hp * _WPAD), (0, 0)))
    xf = xf.astype(jnp.bfloat16)
    xs = [xf[:, k:k + Rpad, :] for k in range(3)]
    w1 = jnp.transpose(W1, (2, 3, 1, 0)).reshape(9, C, dim).astype(jnp.bfloat16)
    w2 = jnp.pad(W2[:, :, 0, 0].T, ((0, 0), (0, CO2 - co))).astype(jnp.bfloat16)
    b1r = b1.reshape(1, dim)
    b2r = jnp.pad(b2, (0, CO2 - co)).reshape(1, CO2)

    body = functools.partial(_rpn_body, rt=RT)
    x_spec = pl.BlockSpec((1, Rpad, C), lambda b, r: (b, 0, 0))
    out = pl.pallas_call(
        body,
        grid=(B, nR),
        in_specs=[
            x_spec, x_spec, x_spec,
            pl.BlockSpec((9, C, dim), lambda b, r: (0, 0, 0)),
            pl.BlockSpec((1, dim), lambda b, r: (0, 0)),
            pl.BlockSpec((dim, CO2), lambda b, r: (0, 0)),
            pl.BlockSpec((1, CO2), lambda b, r: (0, 0)),
        ],
        out_specs=pl.BlockSpec((1, RT, CO2), lambda b, r: (b, r, 0)),
        out_shape=jax.ShapeDtypeStruct((B, nR * RT, CO2), jnp.float32),
    )(*xs, w1, b1r, w2, b2r)

    out = out[:, :H * _WPAD, :].reshape(B, H, _WPAD, CO2)[:, :, :W, :co]
    return out.reshape(B, H, W, co // 6, 6)
